# Initial kernel scaffold; baseline (speedup 1.0000x reference)
import jax
import jax.numpy as jnp
from jax.experimental import pallas as pl


def kernel(x, edge_index, W1, a_s1, a_d1, b1, W2, a_s2, a_d2, b2, lw1, lb1, lw2, lb2):
    raise NotImplementedError("write your pallas kernel here")


# speedup vs baseline: 8.7201x; 8.7201x over previous
import jax
import jax.numpy as jnp
from jax import lax
from jax.experimental import pallas as pl
from jax.experimental.pallas import tpu as pltpu
from jax.experimental.pallas import tpu_sc as plsc

N = 10000
E = 320000
D = 128
H = 16
OUT = 128
HD = H * OUT

NC = 2
NS = 16
NW = NC * NS

NP = 10240
ROWS_PER_TILE = NP // NS
ZB = 128

B1 = 80
NB1 = E // NS // B1
B2 = 16
NB2 = E // NS // B2
ZB2 = 32


def _dense_body(x_ref, W1_ref, W1r_ref, as1_ref, ad1_ref,
                W2_ref, W2r_ref, as2_ref, ad2_ref,
                lw1_ref, lb1_ref, lw2_ref, lb2_ref,
                h1_ref, h2_ref, tab_ref, xs_ref):
    x = x_ref[...]
    h1_ref[...] = lax.dot_general(x, W1_ref[...], (((1,), (1,)), ((), ())),
                                  preferred_element_type=jnp.float32)
    h2_ref[...] = lax.dot_general(x, W2_ref[...], (((1,), (1,)), ((), ())),
                                  preferred_element_type=jnp.float32)
    bdims = (((1,), (1,)), ((0,), (0,)))
    cs1 = lax.dot_general(as1_ref[...], W1r_ref[...], bdims,
                          preferred_element_type=jnp.float32)
    cd1 = lax.dot_general(ad1_ref[...], W1r_ref[...], bdims,
                          preferred_element_type=jnp.float32)
    cs2 = lax.dot_general(as2_ref[...], W2r_ref[...], bdims,
                          preferred_element_type=jnp.float32)
    cd2 = lax.dot_general(ad2_ref[...], W2r_ref[...], bdims,
                          preferred_element_type=jnp.float32)
    nd = (((1,), (1,)), ((), ()))
    asrc1 = lax.dot_general(x, cs1, nd, preferred_element_type=jnp.float32)
    adst1 = lax.dot_general(x, cd1, nd, preferred_element_type=jnp.float32)
    asrc2 = lax.dot_general(x, cs2, nd, preferred_element_type=jnp.float32)
    adst2 = lax.dot_general(x, cd2, nd, preferred_element_type=jnp.float32)
    pad = jnp.zeros((asrc1.shape[0], OUT - 4 * H), jnp.float32)
    tab_ref[...] = jnp.concatenate([asrc1, adst2, adst1, asrc2, pad], axis=1)
    mid = lax.dot_general(x, lw1_ref[...], nd,
                          preferred_element_type=jnp.float32) + lb1_ref[...]
    mid = jnp.where(mid > 0, mid, jnp.exp(jnp.minimum(mid, 0.0)) - 1.0)
    xs = lax.dot_general(mid, lw2_ref[...], nd,
                         preferred_element_type=jnp.float32) + lb2_ref[...]
    xs_ref[...] = jnp.where(xs > 0, xs, jnp.exp(jnp.minimum(xs, 0.0)) - 1.0)


def _dense(x, W1, W1r, as1, ad1, W2, W2r, as2, ad2, lw1, lb1, lw2, lb2):
    RB = 400
    grid = (N // RB,)
    full = lambda shape: pl.BlockSpec(shape, lambda i: tuple(0 for _ in shape))
    row = lambda shape: pl.BlockSpec(shape, lambda i: (i,) + (0,) * (len(shape) - 1))
    return pl.pallas_call(
        _dense_body,
        grid=grid,
        in_specs=[
            row((RB, D)),
            full((HD, D)), full((H, OUT, D)), full((H, OUT)), full((H, OUT)),
            full((HD, D)), full((H, OUT, D)), full((H, OUT)), full((H, OUT)),
            full((4 * OUT, D)), full((4 * OUT,)), full((OUT, 4 * OUT)),
            full((OUT,)),
        ],
        out_specs=[
            row((RB, HD)), row((RB, HD)),
            row((RB, OUT)), row((RB, OUT)),
        ],
        out_shape=[
            jax.ShapeDtypeStruct((N, HD), jnp.float32),
            jax.ShapeDtypeStruct((N, HD), jnp.float32),
            jax.ShapeDtypeStruct((N, OUT), jnp.float32),
            jax.ShapeDtypeStruct((N, OUT), jnp.float32),
        ],
    )(x, W1, W1r, as1, ad1, W2, W2r, as2, ad2, lw1, lb1, lw2, lb2)


def _sc_logits_body(eu, ev, tab,
                    ee1, ee2, dd,
                    idx_u, idx_v, tu_buf, tv_buf, e16_buf, e128_buf,
                    zbuf, d_sh, sem):
    cid = lax.axis_index("c")
    sid = lax.axis_index("s")

    @pl.loop(0, B1)
    def _zpad(i):
        for k in range(OUT // 16):
            e128_buf[i, pl.ds(k * 16, 16)] = jnp.zeros((16,), jnp.float32)

    @pl.loop(0, ZB2)
    def _zero(i):
        for k in range(OUT // 16):
            zbuf[i, pl.ds(k * 16, 16)] = jnp.zeros((16,), jnp.float32)

    row0 = sid * ROWS_PER_TILE
    for q in range(ROWS_PER_TILE // ZB2):
        pltpu.sync_copy(zbuf, d_sh.at[pl.ds(row0 + q * ZB2, ZB2)])
    plsc.subcore_barrier()

    def run(ee_out, co):
        @pl.loop(0, NB1)
        def _batch(j):
            base = sid * (E // NS) + j * B1
            pltpu.sync_copy(eu.at[pl.ds(base, B1)], idx_u)
            pltpu.sync_copy(ev.at[pl.ds(base, B1)], idx_v)
            pltpu.async_copy(tab.at[idx_u], tu_buf, sem).wait()
            pltpu.async_copy(tab.at[idx_v], tv_buf, sem).wait()

            @pl.loop(0, B1)
            def _row(i):
                if co == 0:
                    e = tu_buf[i, 0:16] + tv_buf[i, 32:48]
                else:
                    e = tv_buf[i, 48:64] + tu_buf[i, 16:32]
                e = jnp.where(e >= 0, e, e * 0.2)
                e = jnp.exp(e)
                e16_buf[i, :] = e
                e128_buf[i, pl.ds(co, 16)] = e

            pltpu.sync_copy(e16_buf, ee_out.at[pl.ds(base, B1)])
            if co == 0:
                pltpu.sync_copy(e128_buf, d_sh.at[idx_v], add=True)
            else:
                pltpu.sync_copy(e128_buf, d_sh.at[idx_u], add=True)

    @pl.when(cid == 0)
    def _():
        run(ee1, 0)

    @pl.when(cid == 1)
    def _():
        run(ee2, 16)

    plsc.subcore_barrier()
    pltpu.sync_copy(d_sh.at[pl.ds(row0, ROWS_PER_TILE)],
                    dd.at[pl.ds(cid * NP + row0, ROWS_PER_TILE)])


def _sc_logits(eu, ev, tab):
    mesh = plsc.VectorSubcoreMesh(core_axis_name="c", subcore_axis_name="s",
                                  num_cores=NC, num_subcores=NS)
    f32 = jnp.float32
    return pl.kernel(
        _sc_logits_body,
        out_type=[
            jax.ShapeDtypeStruct((E, H), f32),
            jax.ShapeDtypeStruct((E, H), f32),
            jax.ShapeDtypeStruct((NC * NP, OUT), f32),
        ],
        mesh=mesh,
        scratch_types=[
            pltpu.VMEM((B1,), jnp.int32),
            pltpu.VMEM((B1,), jnp.int32),
            pltpu.VMEM((B1, OUT), f32),
            pltpu.VMEM((B1, OUT), f32),
            pltpu.VMEM((B1, H), f32),
            pltpu.VMEM((B1, OUT), f32),
            pltpu.VMEM((ZB2, OUT), f32),
            pltpu.VMEM_SHARED((NP, OUT), f32),
            pltpu.SemaphoreType.DMA,
        ],
    )(eu, ev, tab)


def _rdenom_body(da, db, rd_ref):
    r1 = 1.0 / (da[:, 0:16] + 1e-16)
    r2 = 1.0 / (db[:, 16:32] + 1e-16)
    pad = jnp.zeros((r1.shape[0], OUT - 2 * H), jnp.float32)
    rd_ref[...] = jnp.concatenate([r1, r2, pad], axis=1)


def _rdenom(dd):
    f32 = jnp.float32
    return pl.pallas_call(
        _rdenom_body,
        out_shape=jax.ShapeDtypeStruct((NP, OUT), f32),
    )(dd[:NP], dd[NP:])


def _sc_agg_body(eu, ev, ee1, ee2, rd, h1, h2,
                 osum,
                 idx_s, idx_d, hbuf, eebuf, rdbuf, msg,
                 zbuf, o_sh, sem):
    cid = lax.axis_index("c")
    sid = lax.axis_index("s")

    @pl.loop(0, ZB2)
    def _zero(i):
        for k in range(OUT // 16):
            zbuf[i, pl.ds(k * 16, 16)] = jnp.zeros((16,), jnp.float32)

    row0 = sid * ROWS_PER_TILE
    for q in range(ROWS_PER_TILE // ZB2):
        pltpu.sync_copy(zbuf, o_sh.at[pl.ds(row0 + q * ZB2, ZB2)])
    plsc.subcore_barrier()

    def run(src_hbm, dst_hbm, ee_hbm, h_hbm, co):
        @pl.loop(0, NB2)
        def _batch(j):
            base = sid * (E // NS) + j * B2
            pltpu.sync_copy(src_hbm.at[pl.ds(base, B2)], idx_s)
            pltpu.sync_copy(dst_hbm.at[pl.ds(base, B2)], idx_d)
            pltpu.async_copy(h_hbm.at[idx_s], hbuf, sem).wait()
            pltpu.async_copy(rd.at[idx_d], rdbuf, sem).wait()
            pltpu.sync_copy(ee_hbm.at[pl.ds(base, B2)], eebuf)

            @pl.loop(0, B2)
            def _edge(i):
                a = eebuf[i, :] * rdbuf[i, pl.ds(co, 16)]
                alphas = [a[h] for h in range(H)]
                for k in range(OUT // 16):
                    acc = alphas[0] * hbuf[i, pl.ds(k * 16, 16)]
                    for h in range(1, H):
                        acc = acc + alphas[h] * hbuf[i, pl.ds(h * OUT + k * 16, 16)]
                    msg[i, pl.ds(k * 16, 16)] = acc

            pltpu.sync_copy(msg, o_sh.at[idx_d], add=True)

    @pl.when(cid == 0)
    def _():
        run(eu, ev, ee1, h1, 0)

    @pl.when(cid == 1)
    def _():
        run(ev, eu, ee2, h2, 16)

    plsc.subcore_barrier()
    pltpu.sync_copy(o_sh.at[pl.ds(row0, ROWS_PER_TILE)],
                    osum.at[pl.ds(cid * NP + row0, ROWS_PER_TILE)])


def _sc_agg(eu, ev, ee1, ee2, rd, h1, h2):
    mesh = plsc.VectorSubcoreMesh(core_axis_name="c", subcore_axis_name="s",
                                  num_cores=NC, num_subcores=NS)
    f32 = jnp.float32
    return pl.kernel(
        _sc_agg_body,
        out_type=jax.ShapeDtypeStruct((NC * NP, OUT), f32),
        mesh=mesh,
        scratch_types=[
            pltpu.VMEM((B2,), jnp.int32),
            pltpu.VMEM((B2,), jnp.int32),
            pltpu.VMEM((B2, HD), f32),
            pltpu.VMEM((B2, H), f32),
            pltpu.VMEM((B2, OUT), f32),
            pltpu.VMEM((B2, OUT), f32),
            pltpu.VMEM((ZB2, OUT), f32),
            pltpu.VMEM_SHARED((NP, OUT), f32),
            pltpu.SemaphoreType.DMA,
        ],
    )(eu, ev, ee1, ee2, rd, h1, h2)


def _finish_body(o1_ref, o2_ref, b1_ref, b2_ref, xin_ref, xout_ref):
    s = 1.0 / H
    a = o1_ref[...] * s + b1_ref[...]
    xin_ref[...] = jnp.where(a > 0, a, jnp.exp(jnp.minimum(a, 0.0)) - 1.0)
    b = o2_ref[...] * s + b2_ref[...]
    xout_ref[...] = jnp.where(b > 0, b, jnp.exp(jnp.minimum(b, 0.0)) - 1.0)


def _finish(o1, o2, b1, b2):
    RB = 1000
    grid = (N // RB,)
    row = pl.BlockSpec((RB, OUT), lambda i: (i, 0))
    vec = pl.BlockSpec((OUT,), lambda i: (0,))
    return pl.pallas_call(
        _finish_body,
        grid=grid,
        in_specs=[row, row, vec, vec],
        out_specs=[row, row],
        out_shape=[
            jax.ShapeDtypeStruct((N, OUT), jnp.float32),
            jax.ShapeDtypeStruct((N, OUT), jnp.float32),
        ],
    )(o1, o2, b1, b2)


@jax.jit
def kernel(x, edge_index, W1, a_s1, a_d1, b1, W2, a_s2, a_d2, b2,
           lw1, lb1, lw2, lb2):
    eu = edge_index[0]
    ev = edge_index[1]
    W1r = W1.reshape(H, OUT, D)
    W2r = W2.reshape(H, OUT, D)
    as1 = a_s1.reshape(H, OUT)
    ad1 = a_d1.reshape(H, OUT)
    as2 = a_s2.reshape(H, OUT)
    ad2 = a_d2.reshape(H, OUT)

    h1, h2, tab, xs = _dense(x, W1, W1r, as1, ad1, W2, W2r, as2, ad2,
                             lw1, lb1, lw2, lb2)
    ee1, ee2, dd = _sc_logits(eu, ev, tab)
    rd = _rdenom(dd)
    osum = _sc_agg(eu, ev, ee1, ee2, rd, h1, h2)
    x_in, x_out = _finish(osum[:N], osum[NP:NP + N], b1, b2)
    return (x_in, x_out, xs)


# speedup vs baseline: 14.8334x; 1.7010x over previous
import jax
import jax.numpy as jnp
from jax import lax
from jax.experimental import pallas as pl
from jax.experimental.pallas import tpu as pltpu
from jax.experimental.pallas import tpu_sc as plsc

N = 10000
E = 320000
D = 128
H = 16
OUT = 128
HD = H * OUT

NC = 2
NS = 16
NW = NC * NS

NP = 10240
ROWS_PER_TILE = NP // NS
ZB = 128

B1 = 64
NB1G = E // B1
TB1 = (NB1G + NS - 1) // NS
B2 = 8
ZB2 = 32


def _dense_body(x_ref, W1_ref, W1r_ref, as1_ref, ad1_ref,
                W2_ref, W2r_ref, as2_ref, ad2_ref,
                lw1_ref, lb1_ref, lw2_ref, lb2_ref,
                h1_ref, h2_ref, tab_ref, xs_ref):
    x = x_ref[...]
    h1_ref[...] = lax.dot_general(x, W1_ref[...], (((1,), (1,)), ((), ())),
                                  preferred_element_type=jnp.float32)
    h2_ref[...] = lax.dot_general(x, W2_ref[...], (((1,), (1,)), ((), ())),
                                  preferred_element_type=jnp.float32)
    bdims = (((1,), (1,)), ((0,), (0,)))
    cs1 = lax.dot_general(as1_ref[...], W1r_ref[...], bdims,
                          preferred_element_type=jnp.float32)
    cd1 = lax.dot_general(ad1_ref[...], W1r_ref[...], bdims,
                          preferred_element_type=jnp.float32)
    cs2 = lax.dot_general(as2_ref[...], W2r_ref[...], bdims,
                          preferred_element_type=jnp.float32)
    cd2 = lax.dot_general(ad2_ref[...], W2r_ref[...], bdims,
                          preferred_element_type=jnp.float32)
    nd = (((1,), (1,)), ((), ()))
    asrc1 = lax.dot_general(x, cs1, nd, preferred_element_type=jnp.float32)
    adst1 = lax.dot_general(x, cd1, nd, preferred_element_type=jnp.float32)
    asrc2 = lax.dot_general(x, cs2, nd, preferred_element_type=jnp.float32)
    adst2 = lax.dot_general(x, cd2, nd, preferred_element_type=jnp.float32)
    pad = jnp.zeros((asrc1.shape[0], OUT - 4 * H), jnp.float32)
    tab_ref[...] = jnp.concatenate([asrc1, adst2, adst1, asrc2, pad], axis=1)
    mid = lax.dot_general(x, lw1_ref[...], nd,
                          preferred_element_type=jnp.float32) + lb1_ref[...]
    mid = jnp.where(mid > 0, mid, jnp.exp(jnp.minimum(mid, 0.0)) - 1.0)
    xs = lax.dot_general(mid, lw2_ref[...], nd,
                         preferred_element_type=jnp.float32) + lb2_ref[...]
    xs_ref[...] = jnp.where(xs > 0, xs, jnp.exp(jnp.minimum(xs, 0.0)) - 1.0)


def _dense(x, W1, W1r, as1, ad1, W2, W2r, as2, ad2, lw1, lb1, lw2, lb2):
    RB = 400
    grid = (N // RB,)
    full = lambda shape: pl.BlockSpec(shape, lambda i: tuple(0 for _ in shape))
    row = lambda shape: pl.BlockSpec(shape, lambda i: (i,) + (0,) * (len(shape) - 1))
    return pl.pallas_call(
        _dense_body,
        grid=grid,
        in_specs=[
            row((RB, D)),
            full((HD, D)), full((H, OUT, D)), full((H, OUT)), full((H, OUT)),
            full((HD, D)), full((H, OUT, D)), full((H, OUT)), full((H, OUT)),
            full((4 * OUT, D)), full((4 * OUT,)), full((OUT, 4 * OUT)),
            full((OUT,)),
        ],
        out_specs=[
            row((RB, HD)), row((RB, HD)),
            row((RB, OUT)), row((RB, OUT)),
        ],
        out_shape=[
            jax.ShapeDtypeStruct((N, HD), jnp.float32),
            jax.ShapeDtypeStruct((N, HD), jnp.float32),
            jax.ShapeDtypeStruct((N, OUT), jnp.float32),
            jax.ShapeDtypeStruct((N, OUT), jnp.float32),
        ],
    )(x, W1, W1r, as1, ad1, W2, W2r, as2, ad2, lw1, lb1, lw2, lb2)


def _sc_logits_body(eu, ev, tab,
                    ee1, ee2, dd,
                    idx_u, idx_v, tu_buf, tv_buf, e16_buf, e128_buf,
                    zbuf, d_sh, sem):
    cid = lax.axis_index("c")
    sid = lax.axis_index("s")

    @pl.loop(0, B1)
    def _zpad(i):
        for k in range(OUT // 16):
            e128_buf[i, pl.ds(k * 16, 16)] = jnp.zeros((16,), jnp.float32)

    @pl.loop(0, ZB2)
    def _zero(i):
        for k in range(OUT // 16):
            zbuf[i, pl.ds(k * 16, 16)] = jnp.zeros((16,), jnp.float32)

    row0 = pl.multiple_of(sid * ROWS_PER_TILE, ROWS_PER_TILE)
    for q in range(ROWS_PER_TILE // ZB2):
        pltpu.sync_copy(zbuf, d_sh.at[pl.ds(row0 + q * ZB2, ZB2)])
    plsc.subcore_barrier()

    def run(ee_out, co):
        @pl.loop(0, TB1)
        def _batch(j):
            g = j * NS + sid

            @pl.when(g < NB1G)
            def _():
                base = pl.multiple_of(g * B1, B1)
                pltpu.sync_copy(eu.at[pl.ds(base, B1)], idx_u)
                pltpu.sync_copy(ev.at[pl.ds(base, B1)], idx_v)
                pltpu.async_copy(tab.at[idx_u], tu_buf, sem).wait()
                pltpu.async_copy(tab.at[idx_v], tv_buf, sem).wait()

                @pl.loop(0, B1 // 8)
                def _row(r):
                    for q in range(8):
                        i = r * 8 + q
                        if co == 0:
                            e = tu_buf[i, 0:16] + tv_buf[i, 32:48]
                        else:
                            e = tv_buf[i, 48:64] + tu_buf[i, 16:32]
                        e = jnp.where(e >= 0, e, e * 0.2)
                        e = jnp.exp(e)
                        e16_buf[r, pl.ds(q * 16, 16)] = e
                        e128_buf[i, pl.ds(co, 16)] = e

                pltpu.sync_copy(e16_buf, ee_out.at[pl.ds(pl.multiple_of(base // 8, 8), B1 // 8)])
                if co == 0:
                    pltpu.sync_copy(e128_buf, d_sh.at[idx_v], add=True)
                else:
                    pltpu.sync_copy(e128_buf, d_sh.at[idx_u], add=True)

    @pl.when(cid == 0)
    def _():
        run(ee1, 0)

    @pl.when(cid == 1)
    def _():
        run(ee2, 16)

    plsc.subcore_barrier()
    pltpu.sync_copy(d_sh.at[pl.ds(row0, ROWS_PER_TILE)],
                    dd.at[pl.ds(pl.multiple_of(cid * NP + row0, ROWS_PER_TILE), ROWS_PER_TILE)])


def _sc_logits(eu, ev, tab):
    mesh = plsc.VectorSubcoreMesh(core_axis_name="c", subcore_axis_name="s",
                                  num_cores=NC, num_subcores=NS)
    f32 = jnp.float32
    return pl.kernel(
        _sc_logits_body,
        out_type=[
            jax.ShapeDtypeStruct((E // 8, 128), f32),
            jax.ShapeDtypeStruct((E // 8, 128), f32),
            jax.ShapeDtypeStruct((NC * NP, OUT), f32),
        ],
        mesh=mesh,
        scratch_types=[
            pltpu.VMEM((B1,), jnp.int32),
            pltpu.VMEM((B1,), jnp.int32),
            pltpu.VMEM((B1, OUT), f32),
            pltpu.VMEM((B1, OUT), f32),
            pltpu.VMEM((B1 // 8, 128), f32),
            pltpu.VMEM((B1, OUT), f32),
            pltpu.VMEM((ZB2, OUT), f32),
            pltpu.VMEM_SHARED((NP, OUT), f32),
            pltpu.SemaphoreType.DMA,
        ],
    )(eu, ev, tab)


def _rdenom_body(da, db, rd_ref):
    r1 = 1.0 / (da[:, 0:16] + 1e-16)
    r2 = 1.0 / (db[:, 16:32] + 1e-16)
    pad = jnp.zeros((r1.shape[0], OUT - 2 * H), jnp.float32)
    rd_ref[...] = jnp.concatenate([r1, r2, pad], axis=1)


def _rdenom(dd):
    f32 = jnp.float32
    return pl.pallas_call(
        _rdenom_body,
        out_shape=jax.ShapeDtypeStruct((NP, OUT), f32),
    )(dd[:NP], dd[NP:])


CH = 64
NCHG = E // (CH * B2)
TCH = (NCHG + NS - 1) // NS


def _sc_agg_body(eu, ev, ee1, ee2, rd, h1, h2,
                 osum,
                 idxs_c, idxd_c, ee_c, hbufs, rdbufs, msg,
                 o_sh, semh0, semh1, semr0, semr1):
    cid = lax.axis_index("c")
    sid = lax.axis_index("s")
    hbuf = [hbufs.at[0], hbufs.at[1]]
    rdbuf = [rdbufs.at[0], rdbufs.at[1]]
    semh = [semh0, semh1]
    semr = [semr0, semr1]

    @pl.loop(0, 2 * B2)
    def _zm(i):
        for k in range(OUT // 16):
            msg[i, pl.ds(k * 16, 16)] = jnp.zeros((16,), jnp.float32)

    row0 = pl.multiple_of(sid * ROWS_PER_TILE, ROWS_PER_TILE)

    @pl.loop(0, ROWS_PER_TILE // (2 * B2))
    def _zs(q):
        pltpu.sync_copy(msg, o_sh.at[pl.ds(row0 + q * 2 * B2, 2 * B2)])
    plsc.subcore_barrier()

    def run(src_hbm, dst_hbm, ee_hbm, h_hbm, co):
        def issue(b, p):
            pltpu.async_copy(h_hbm.at[idxs_c.at[pl.ds(b * B2, B2)]],
                             hbuf[p], semh[p])
            pltpu.async_copy(rd.at[idxd_c.at[pl.ds(b * B2, B2)]],
                             rdbuf[p], semr[p])

        def drain(p):
            pltpu.make_async_copy(h_hbm.at[pl.ds(0, B2)], hbuf[p], semh[p]).wait()
            pltpu.make_async_copy(rd.at[pl.ds(0, B2)], rdbuf[p], semr[p]).wait()

        def compute(b, p, mrow):
            for i in range(B2):
                a = ee_c[b, pl.ds(i * 16, 16)] * rdbuf[p][i, pl.ds(co, 16)]
                alphas = [a[h] for h in range(H)]
                for k in range(OUT // 16):
                    acc = alphas[0] * hbuf[p][i, pl.ds(k * 16, 16)]
                    for h in range(1, H):
                        acc = acc + alphas[h] * hbuf[p][i, pl.ds(h * OUT + k * 16, 16)]
                    msg[mrow + i, pl.ds(k * 16, 16)] = acc

        @pl.loop(0, TCH)
        def _chunk(c):
            cglob = c * NS + sid

            @pl.when(cglob < NCHG)
            def _():
                base = pl.multiple_of(cglob * CH * B2, CH * B2)
                pltpu.sync_copy(src_hbm.at[pl.ds(base, CH * B2)], idxs_c)
                pltpu.sync_copy(dst_hbm.at[pl.ds(base, CH * B2)], idxd_c)
                pltpu.sync_copy(ee_hbm.at[pl.ds(pl.multiple_of(base // 8, CH), CH)], ee_c)
                issue(0, 0)
                issue(1, 1)

                @pl.loop(0, CH // 2)
                def _pair(t):
                    b0 = t * 2
                    drain(0)
                    compute(b0, 0, 0)
                    @pl.when(b0 + 2 < CH)
                    def _i0():
                        issue(b0 + 2, 0)
                    drain(1)
                    compute(b0 + 1, 1, B2)
                    @pl.when(b0 + 3 < CH)
                    def _i1():
                        issue(b0 + 3, 1)
                    idxv = idxd_c[pl.ds(b0 * B2, 2 * B2)]
                    pltpu.sync_copy(msg, o_sh.at[idxv], add=True)

    @pl.when(cid == 0)
    def _():
        run(eu, ev, ee1, h1, 0)

    @pl.when(cid == 1)
    def _():
        run(ev, eu, ee2, h2, 16)

    plsc.subcore_barrier()
    pltpu.sync_copy(o_sh.at[pl.ds(row0, ROWS_PER_TILE)],
                    osum.at[pl.ds(pl.multiple_of(cid * NP + row0, ROWS_PER_TILE), ROWS_PER_TILE)])


def _sc_agg(eu, ev, ee1, ee2, rd, h1, h2):
    mesh = plsc.VectorSubcoreMesh(core_axis_name="c", subcore_axis_name="s",
                                  num_cores=NC, num_subcores=NS)
    f32 = jnp.float32
    return pl.kernel(
        _sc_agg_body,
        out_type=jax.ShapeDtypeStruct((NC * NP, OUT), f32),
        mesh=mesh,
        scratch_types=[
            pltpu.VMEM((CH * B2,), jnp.int32),
            pltpu.VMEM((CH * B2,), jnp.int32),
            pltpu.VMEM((CH, 128), f32),
            pltpu.VMEM((2, B2, HD), f32),
            pltpu.VMEM((2, B2, OUT), f32),
            pltpu.VMEM((2 * B2, OUT), f32),
            pltpu.VMEM_SHARED((NP, OUT), f32),
            pltpu.SemaphoreType.DMA,
            pltpu.SemaphoreType.DMA,
            pltpu.SemaphoreType.DMA,
            pltpu.SemaphoreType.DMA,
        ],
    )(eu, ev, ee1, ee2, rd, h1, h2)


def _finish_body(o1_ref, o2_ref, b1_ref, b2_ref, xin_ref, xout_ref):
    s = 1.0 / H
    a = o1_ref[...] * s + b1_ref[...]
    xin_ref[...] = jnp.where(a > 0, a, jnp.exp(jnp.minimum(a, 0.0)) - 1.0)
    b = o2_ref[...] * s + b2_ref[...]
    xout_ref[...] = jnp.where(b > 0, b, jnp.exp(jnp.minimum(b, 0.0)) - 1.0)


def _finish(o1, o2, b1, b2):
    RB = 1000
    grid = (N // RB,)
    row = pl.BlockSpec((RB, OUT), lambda i: (i, 0))
    vec = pl.BlockSpec((OUT,), lambda i: (0,))
    return pl.pallas_call(
        _finish_body,
        grid=grid,
        in_specs=[row, row, vec, vec],
        out_specs=[row, row],
        out_shape=[
            jax.ShapeDtypeStruct((N, OUT), jnp.float32),
            jax.ShapeDtypeStruct((N, OUT), jnp.float32),
        ],
    )(o1, o2, b1, b2)


@jax.jit
def kernel(x, edge_index, W1, a_s1, a_d1, b1, W2, a_s2, a_d2, b2,
           lw1, lb1, lw2, lb2):
    eu = edge_index[0]
    ev = edge_index[1]
    W1r = W1.reshape(H, OUT, D)
    W2r = W2.reshape(H, OUT, D)
    as1 = a_s1.reshape(H, OUT)
    ad1 = a_d1.reshape(H, OUT)
    as2 = a_s2.reshape(H, OUT)
    ad2 = a_d2.reshape(H, OUT)

    h1, h2, tab, xs = _dense(x, W1, W1r, as1, ad1, W2, W2r, as2, ad2,
                             lw1, lb1, lw2, lb2)
    ee1, ee2, dd = _sc_logits(eu, ev, tab)
    rd = _rdenom(dd)
    osum = _sc_agg(eu, ev, ee1, ee2, rd, h1, h2)
    x_in, x_out = _finish(osum[:N], osum[NP:NP + N], b1, b2)
    return (x_in, x_out, xs)


# speedup vs baseline: 16.0265x; 1.0804x over previous
import jax
import jax.numpy as jnp
from jax import lax
from jax.experimental import pallas as pl
from jax.experimental.pallas import tpu as pltpu
from jax.experimental.pallas import tpu_sc as plsc

N = 10000
E = 320000
D = 128
H = 16
OUT = 128
HD = H * OUT

NC = 2
NS = 16
NW = NC * NS

NP = 10240
ROWS_PER_TILE = NP // NS
ZB = 128

B1 = 64
NB1G = E // B1
TB1 = (NB1G + NW - 1) // NW
B2 = 8
ZB2 = 32


def _dense_body(x_ref, W1_ref, W1r_ref, as1_ref, ad1_ref,
                W2_ref, W2r_ref, as2_ref, ad2_ref,
                lw1_ref, lb1_ref, lw2_ref, lb2_ref,
                h1_ref, h2_ref, tab_ref, xs_ref):
    x = x_ref[...]
    h1_ref[...] = lax.dot_general(x, W1_ref[...], (((1,), (1,)), ((), ())),
                                  preferred_element_type=jnp.float32)
    h2_ref[...] = lax.dot_general(x, W2_ref[...], (((1,), (1,)), ((), ())),
                                  preferred_element_type=jnp.float32)
    bdims = (((1,), (1,)), ((0,), (0,)))
    cs1 = lax.dot_general(as1_ref[...], W1r_ref[...], bdims,
                          preferred_element_type=jnp.float32)
    cd1 = lax.dot_general(ad1_ref[...], W1r_ref[...], bdims,
                          preferred_element_type=jnp.float32)
    cs2 = lax.dot_general(as2_ref[...], W2r_ref[...], bdims,
                          preferred_element_type=jnp.float32)
    cd2 = lax.dot_general(ad2_ref[...], W2r_ref[...], bdims,
                          preferred_element_type=jnp.float32)
    nd = (((1,), (1,)), ((), ()))
    asrc1 = lax.dot_general(x, cs1, nd, preferred_element_type=jnp.float32)
    adst1 = lax.dot_general(x, cd1, nd, preferred_element_type=jnp.float32)
    asrc2 = lax.dot_general(x, cs2, nd, preferred_element_type=jnp.float32)
    adst2 = lax.dot_general(x, cd2, nd, preferred_element_type=jnp.float32)
    pad = jnp.zeros((asrc1.shape[0], OUT - 4 * H), jnp.float32)
    tab_ref[...] = jnp.concatenate([asrc1, adst2, adst1, asrc2, pad], axis=1)
    mid = lax.dot_general(x, lw1_ref[...], nd,
                          preferred_element_type=jnp.float32) + lb1_ref[...]
    mid = jnp.where(mid > 0, mid, jnp.exp(jnp.minimum(mid, 0.0)) - 1.0)
    xs = lax.dot_general(mid, lw2_ref[...], nd,
                         preferred_element_type=jnp.float32) + lb2_ref[...]
    xs_ref[...] = jnp.where(xs > 0, xs, jnp.exp(jnp.minimum(xs, 0.0)) - 1.0)


def _dense(x, W1, W1r, as1, ad1, W2, W2r, as2, ad2, lw1, lb1, lw2, lb2):
    RB = 400
    grid = (N // RB,)
    full = lambda shape: pl.BlockSpec(shape, lambda i: tuple(0 for _ in shape))
    row = lambda shape: pl.BlockSpec(shape, lambda i: (i,) + (0,) * (len(shape) - 1))
    return pl.pallas_call(
        _dense_body,
        grid=grid,
        in_specs=[
            row((RB, D)),
            full((HD, D)), full((H, OUT, D)), full((H, OUT)), full((H, OUT)),
            full((HD, D)), full((H, OUT, D)), full((H, OUT)), full((H, OUT)),
            full((4 * OUT, D)), full((4 * OUT,)), full((OUT, 4 * OUT)),
            full((OUT,)),
        ],
        out_specs=[
            row((RB, HD)), row((RB, HD)),
            row((RB, OUT)), row((RB, OUT)),
        ],
        out_shape=[
            jax.ShapeDtypeStruct((N, HD), jnp.float32),
            jax.ShapeDtypeStruct((N, HD), jnp.float32),
            jax.ShapeDtypeStruct((N, OUT), jnp.float32),
            jax.ShapeDtypeStruct((N, OUT), jnp.float32),
        ],
    )(x, W1, W1r, as1, ad1, W2, W2r, as2, ad2, lw1, lb1, lw2, lb2)


def _sc_logits_body(eu, ev, tab,
                    ee1, ee2, dd,
                    idx_u, idx_v, tu_buf, tv_buf, e16a, e16b,
                    e128a, e128b, d_sh, sem):
    cid = lax.axis_index("c")
    sid = lax.axis_index("s")
    wid = cid * NS + sid

    @pl.loop(0, B1)
    def _zpad(i):
        for k in range(OUT // 16):
            e128a[i, pl.ds(k * 16, 16)] = jnp.zeros((16,), jnp.float32)
            e128b[i, pl.ds(k * 16, 16)] = jnp.zeros((16,), jnp.float32)

    row0 = pl.multiple_of(sid * ROWS_PER_TILE, ROWS_PER_TILE)

    @pl.loop(0, ROWS_PER_TILE // B1)
    def _zs(q):
        pltpu.sync_copy(e128a, d_sh.at[pl.ds(row0 + q * B1, B1)])
    plsc.subcore_barrier()

    @pl.loop(0, TB1)
    def _batch(j):
        g = j * NW + wid

        @pl.when(g < NB1G)
        def _():
            base = pl.multiple_of(g * B1, B1)
            pltpu.sync_copy(eu.at[pl.ds(base, B1)], idx_u)
            pltpu.sync_copy(ev.at[pl.ds(base, B1)], idx_v)
            pltpu.async_copy(tab.at[idx_u], tu_buf, sem).wait()
            pltpu.async_copy(tab.at[idx_v], tv_buf, sem).wait()

            @pl.loop(0, B1 // 8)
            def _row(r):
                for q in range(8):
                    i = r * 8 + q
                    e1 = tu_buf[i, 0:16] + tv_buf[i, 32:48]
                    e1 = jnp.where(e1 >= 0, e1, e1 * 0.2)
                    e1 = jnp.exp(e1)
                    e16a[r, pl.ds(q * 16, 16)] = e1
                    e128a[i, 0:16] = e1
                    e2 = tv_buf[i, 48:64] + tu_buf[i, 16:32]
                    e2 = jnp.where(e2 >= 0, e2, e2 * 0.2)
                    e2 = jnp.exp(e2)
                    e16b[r, pl.ds(q * 16, 16)] = e2
                    e128b[i, pl.ds(16, 16)] = e2

            row_off = pl.multiple_of(base // 8, 8)
            pltpu.sync_copy(e16a, ee1.at[pl.ds(row_off, B1 // 8)])
            pltpu.sync_copy(e16b, ee2.at[pl.ds(row_off, B1 // 8)])
            pltpu.sync_copy(e128a, d_sh.at[idx_v], add=True)
            pltpu.sync_copy(e128b, d_sh.at[idx_u], add=True)

    plsc.subcore_barrier()
    pltpu.sync_copy(d_sh.at[pl.ds(row0, ROWS_PER_TILE)],
                    dd.at[pl.ds(pl.multiple_of(cid * NP + row0, ROWS_PER_TILE), ROWS_PER_TILE)])


def _sc_logits(eu, ev, tab):
    mesh = plsc.VectorSubcoreMesh(core_axis_name="c", subcore_axis_name="s",
                                  num_cores=NC, num_subcores=NS)
    f32 = jnp.float32
    return pl.kernel(
        _sc_logits_body,
        out_type=[
            jax.ShapeDtypeStruct((E // 8, 128), f32),
            jax.ShapeDtypeStruct((E // 8, 128), f32),
            jax.ShapeDtypeStruct((NC * NP, OUT), f32),
        ],
        mesh=mesh,
        scratch_types=[
            pltpu.VMEM((B1,), jnp.int32),
            pltpu.VMEM((B1,), jnp.int32),
            pltpu.VMEM((B1, OUT), f32),
            pltpu.VMEM((B1, OUT), f32),
            pltpu.VMEM((B1 // 8, 128), f32),
            pltpu.VMEM((B1 // 8, 128), f32),
            pltpu.VMEM((B1, OUT), f32),
            pltpu.VMEM((B1, OUT), f32),
            pltpu.VMEM_SHARED((NP, OUT), f32),
            pltpu.SemaphoreType.DMA,
        ],
    )(eu, ev, tab)


def _rdenom_body(da, db, rd_ref):
    r1 = 1.0 / (da[:, 0:16] + db[:, 0:16] + 1e-16)
    r2 = 1.0 / (da[:, 16:32] + db[:, 16:32] + 1e-16)
    pad = jnp.zeros((r1.shape[0], OUT - 2 * H), jnp.float32)
    rd_ref[...] = jnp.concatenate([r1, r2, pad], axis=1)


def _rdenom(dd):
    f32 = jnp.float32
    return pl.pallas_call(
        _rdenom_body,
        out_shape=jax.ShapeDtypeStruct((NP, OUT), f32),
    )(dd[:NP], dd[NP:])


CH = 64
NCHG = E // (CH * B2)
TCH = (NCHG + NS - 1) // NS


def _sc_agg_body(eu, ev, ee1, ee2, rd, h1, h2,
                 osum,
                 idxs_c, idxd_c, ee_c, hbufs, rdbufs, msg,
                 o_sh, semh0, semh1, semr0, semr1):
    cid = lax.axis_index("c")
    sid = lax.axis_index("s")
    hbuf = [hbufs.at[0], hbufs.at[1]]
    rdbuf = [rdbufs.at[0], rdbufs.at[1]]
    semh = [semh0, semh1]
    semr = [semr0, semr1]

    @pl.loop(0, 2 * B2)
    def _zm(i):
        for k in range(OUT // 16):
            msg[i, pl.ds(k * 16, 16)] = jnp.zeros((16,), jnp.float32)

    row0 = pl.multiple_of(sid * ROWS_PER_TILE, ROWS_PER_TILE)

    @pl.loop(0, ROWS_PER_TILE // (2 * B2))
    def _zs(q):
        pltpu.sync_copy(msg, o_sh.at[pl.ds(row0 + q * 2 * B2, 2 * B2)])
    plsc.subcore_barrier()

    def run(src_hbm, dst_hbm, ee_hbm, h_hbm, co):
        def issue(b, p):
            pltpu.async_copy(h_hbm.at[idxs_c.at[pl.ds(b * B2, B2)]],
                             hbuf[p], semh[p])
            pltpu.async_copy(rd.at[idxd_c.at[pl.ds(b * B2, B2)]],
                             rdbuf[p], semr[p])

        def drain(p):
            pltpu.make_async_copy(h_hbm.at[pl.ds(0, B2)], hbuf[p], semh[p]).wait()
            pltpu.make_async_copy(rd.at[pl.ds(0, B2)], rdbuf[p], semr[p]).wait()

        def compute(b, p, mrow):
            for i in range(B2):
                a = ee_c[b, pl.ds(i * 16, 16)] * rdbuf[p][i, pl.ds(co, 16)]
                alphas = [a[h] for h in range(H)]
                for k in range(OUT // 16):
                    acc = alphas[0] * hbuf[p][i, pl.ds(k * 16, 16)]
                    for h in range(1, H):
                        acc = acc + alphas[h] * hbuf[p][i, pl.ds(h * OUT + k * 16, 16)]
                    msg[mrow + i, pl.ds(k * 16, 16)] = acc

        @pl.loop(0, TCH)
        def _chunk(c):
            cglob = c * NS + sid

            @pl.when(cglob < NCHG)
            def _():
                base = pl.multiple_of(cglob * CH * B2, CH * B2)
                pltpu.sync_copy(src_hbm.at[pl.ds(base, CH * B2)], idxs_c)
                pltpu.sync_copy(dst_hbm.at[pl.ds(base, CH * B2)], idxd_c)
                pltpu.sync_copy(ee_hbm.at[pl.ds(pl.multiple_of(base // 8, CH), CH)], ee_c)
                issue(0, 0)
                issue(1, 1)

                @pl.loop(0, CH // 2)
                def _pair(t):
                    b0 = t * 2
                    drain(0)
                    compute(b0, 0, 0)
                    @pl.when(b0 + 2 < CH)
                    def _i0():
                        issue(b0 + 2, 0)
                    drain(1)
                    compute(b0 + 1, 1, B2)
                    @pl.when(b0 + 3 < CH)
                    def _i1():
                        issue(b0 + 3, 1)
                    idxv = idxd_c[pl.ds(b0 * B2, 2 * B2)]
                    pltpu.sync_copy(msg, o_sh.at[idxv], add=True)

    @pl.when(cid == 0)
    def _():
        run(eu, ev, ee1, h1, 0)

    @pl.when(cid == 1)
    def _():
        run(ev, eu, ee2, h2, 16)

    plsc.subcore_barrier()
    pltpu.sync_copy(o_sh.at[pl.ds(row0, ROWS_PER_TILE)],
                    osum.at[pl.ds(pl.multiple_of(cid * NP + row0, ROWS_PER_TILE), ROWS_PER_TILE)])


def _sc_agg(eu, ev, ee1, ee2, rd, h1, h2):
    mesh = plsc.VectorSubcoreMesh(core_axis_name="c", subcore_axis_name="s",
                                  num_cores=NC, num_subcores=NS)
    f32 = jnp.float32
    return pl.kernel(
        _sc_agg_body,
        out_type=jax.ShapeDtypeStruct((NC * NP, OUT), f32),
        mesh=mesh,
        scratch_types=[
            pltpu.VMEM((CH * B2,), jnp.int32),
            pltpu.VMEM((CH * B2,), jnp.int32),
            pltpu.VMEM((CH, 128), f32),
            pltpu.VMEM((2, B2, HD), f32),
            pltpu.VMEM((2, B2, OUT), f32),
            pltpu.VMEM((2 * B2, OUT), f32),
            pltpu.VMEM_SHARED((NP, OUT), f32),
            pltpu.SemaphoreType.DMA,
            pltpu.SemaphoreType.DMA,
            pltpu.SemaphoreType.DMA,
            pltpu.SemaphoreType.DMA,
        ],
    )(eu, ev, ee1, ee2, rd, h1, h2)


def _finish_body(o1_ref, o2_ref, b1_ref, b2_ref, xin_ref, xout_ref):
    s = 1.0 / H
    a = o1_ref[...] * s + b1_ref[...]
    xin_ref[...] = jnp.where(a > 0, a, jnp.exp(jnp.minimum(a, 0.0)) - 1.0)
    b = o2_ref[...] * s + b2_ref[...]
    xout_ref[...] = jnp.where(b > 0, b, jnp.exp(jnp.minimum(b, 0.0)) - 1.0)


def _finish(o1, o2, b1, b2):
    RB = 1000
    grid = (N // RB,)
    row = pl.BlockSpec((RB, OUT), lambda i: (i, 0))
    vec = pl.BlockSpec((OUT,), lambda i: (0,))
    return pl.pallas_call(
        _finish_body,
        grid=grid,
        in_specs=[row, row, vec, vec],
        out_specs=[row, row],
        out_shape=[
            jax.ShapeDtypeStruct((N, OUT), jnp.float32),
            jax.ShapeDtypeStruct((N, OUT), jnp.float32),
        ],
    )(o1, o2, b1, b2)


@jax.jit
def kernel(x, edge_index, W1, a_s1, a_d1, b1, W2, a_s2, a_d2, b2,
           lw1, lb1, lw2, lb2):
    eu = edge_index[0]
    ev = edge_index[1]
    W1r = W1.reshape(H, OUT, D)
    W2r = W2.reshape(H, OUT, D)
    as1 = a_s1.reshape(H, OUT)
    ad1 = a_d1.reshape(H, OUT)
    as2 = a_s2.reshape(H, OUT)
    ad2 = a_d2.reshape(H, OUT)

    h1, h2, tab, xs = _dense(x, W1, W1r, as1, ad1, W2, W2r, as2, ad2,
                             lw1, lb1, lw2, lb2)
    ee1, ee2, dd = _sc_logits(eu, ev, tab)
    rd = _rdenom(dd)
    osum = _sc_agg(eu, ev, ee1, ee2, rd, h1, h2)
    x_in, x_out = _finish(osum[:N], osum[NP:NP + N], b1, b2)
    return (x_in, x_out, xs)


# speedup vs baseline: 16.3005x; 1.0171x over previous
import jax
import jax.numpy as jnp
from jax import lax
from jax.experimental import pallas as pl
from jax.experimental.pallas import tpu as pltpu
from jax.experimental.pallas import tpu_sc as plsc

N = 10000
E = 320000
D = 128
H = 16
OUT = 128
HD = H * OUT

NC = 2
NS = 16
NW = NC * NS

NP = 10240
ROWS_PER_TILE = NP // NS
ZB = 128

B1 = 64
NB1G = E // B1
TB1 = (NB1G + NW - 1) // NW
B2 = 8
ZB2 = 32


def _dense_body(x_ref, W1_ref, W1r_ref, as1_ref, ad1_ref,
                W2_ref, W2r_ref, as2_ref, ad2_ref,
                lw1_ref, lb1_ref, lw2_ref, lb2_ref,
                h1_ref, h2_ref, tab_ref, xs_ref):
    x = x_ref[...]
    h1_ref[...] = lax.dot_general(x, W1_ref[...], (((1,), (1,)), ((), ())),
                                  preferred_element_type=jnp.float32)
    h2_ref[...] = lax.dot_general(x, W2_ref[...], (((1,), (1,)), ((), ())),
                                  preferred_element_type=jnp.float32)
    bdims = (((1,), (1,)), ((0,), (0,)))
    cs1 = lax.dot_general(as1_ref[...], W1r_ref[...], bdims,
                          preferred_element_type=jnp.float32)
    cd1 = lax.dot_general(ad1_ref[...], W1r_ref[...], bdims,
                          preferred_element_type=jnp.float32)
    cs2 = lax.dot_general(as2_ref[...], W2r_ref[...], bdims,
                          preferred_element_type=jnp.float32)
    cd2 = lax.dot_general(ad2_ref[...], W2r_ref[...], bdims,
                          preferred_element_type=jnp.float32)
    nd = (((1,), (1,)), ((), ()))
    asrc1 = lax.dot_general(x, cs1, nd, preferred_element_type=jnp.float32)
    adst1 = lax.dot_general(x, cd1, nd, preferred_element_type=jnp.float32)
    asrc2 = lax.dot_general(x, cs2, nd, preferred_element_type=jnp.float32)
    adst2 = lax.dot_general(x, cd2, nd, preferred_element_type=jnp.float32)
    pad = jnp.zeros((asrc1.shape[0], OUT - 4 * H), jnp.float32)
    tab_ref[...] = jnp.concatenate([asrc1, adst2, adst1, asrc2, pad], axis=1)
    mid = lax.dot_general(x, lw1_ref[...], nd,
                          preferred_element_type=jnp.float32) + lb1_ref[...]
    mid = jnp.where(mid > 0, mid, jnp.exp(jnp.minimum(mid, 0.0)) - 1.0)
    xs = lax.dot_general(mid, lw2_ref[...], nd,
                         preferred_element_type=jnp.float32) + lb2_ref[...]
    xs_ref[...] = jnp.where(xs > 0, xs, jnp.exp(jnp.minimum(xs, 0.0)) - 1.0)


def _dense(x, W1, W1r, as1, ad1, W2, W2r, as2, ad2, lw1, lb1, lw2, lb2):
    RB = 400
    grid = (N // RB,)
    full = lambda shape: pl.BlockSpec(shape, lambda i: tuple(0 for _ in shape))
    row = lambda shape: pl.BlockSpec(shape, lambda i: (i,) + (0,) * (len(shape) - 1))
    return pl.pallas_call(
        _dense_body,
        grid=grid,
        in_specs=[
            row((RB, D)),
            full((HD, D)), full((H, OUT, D)), full((H, OUT)), full((H, OUT)),
            full((HD, D)), full((H, OUT, D)), full((H, OUT)), full((H, OUT)),
            full((4 * OUT, D)), full((4 * OUT,)), full((OUT, 4 * OUT)),
            full((OUT,)),
        ],
        out_specs=[
            row((RB, HD)), row((RB, HD)),
            row((RB, OUT)), row((RB, OUT)),
        ],
        out_shape=[
            jax.ShapeDtypeStruct((N, HD), jnp.float32),
            jax.ShapeDtypeStruct((N, HD), jnp.float32),
            jax.ShapeDtypeStruct((N, OUT), jnp.float32),
            jax.ShapeDtypeStruct((N, OUT), jnp.float32),
        ],
    )(x, W1, W1r, as1, ad1, W2, W2r, as2, ad2, lw1, lb1, lw2, lb2)


def _sc_logits_body(eu, ev, tab,
                    ee1, ee2, dd,
                    idx_u, idx_v, tu_buf, tv_buf, e16a, e16b,
                    e128a, e128b, d_sh, sem):
    cid = lax.axis_index("c")
    sid = lax.axis_index("s")
    wid = cid * NS + sid

    @pl.loop(0, B1)
    def _zpad(i):
        for k in range(OUT // 16):
            e128a[i, pl.ds(k * 16, 16)] = jnp.zeros((16,), jnp.float32)
            e128b[i, pl.ds(k * 16, 16)] = jnp.zeros((16,), jnp.float32)

    row0 = pl.multiple_of(sid * ROWS_PER_TILE, ROWS_PER_TILE)

    @pl.loop(0, ROWS_PER_TILE // B1)
    def _zs(q):
        pltpu.sync_copy(e128a, d_sh.at[pl.ds(row0 + q * B1, B1)])
    plsc.subcore_barrier()

    @pl.loop(0, TB1)
    def _batch(j):
        g = j * NW + wid

        @pl.when(g < NB1G)
        def _():
            base = pl.multiple_of(g * B1, B1)
            pltpu.sync_copy(eu.at[pl.ds(base, B1)], idx_u)
            pltpu.sync_copy(ev.at[pl.ds(base, B1)], idx_v)
            du = pltpu.async_copy(tab.at[idx_u], tu_buf, sem)
            dv = pltpu.async_copy(tab.at[idx_v], tv_buf, sem)
            du.wait()
            dv.wait()

            @pl.loop(0, B1 // 8)
            def _row(r):
                for q in range(8):
                    i = r * 8 + q
                    e1 = tu_buf[i, 0:16] + tv_buf[i, 32:48]
                    e1 = jnp.where(e1 >= 0, e1, e1 * 0.2)
                    e1 = jnp.exp(e1)
                    e16a[r, pl.ds(q * 16, 16)] = e1
                    e128a[i, 0:16] = e1
                    e2 = tv_buf[i, 48:64] + tu_buf[i, 16:32]
                    e2 = jnp.where(e2 >= 0, e2, e2 * 0.2)
                    e2 = jnp.exp(e2)
                    e16b[r, pl.ds(q * 16, 16)] = e2
                    e128b[i, pl.ds(16, 16)] = e2

            row_off = pl.multiple_of(base // 8, 8)
            pltpu.sync_copy(e16a, ee1.at[pl.ds(row_off, B1 // 8)])
            pltpu.sync_copy(e16b, ee2.at[pl.ds(row_off, B1 // 8)])
            pltpu.sync_copy(e128a, d_sh.at[idx_v], add=True)
            pltpu.sync_copy(e128b, d_sh.at[idx_u], add=True)

    plsc.subcore_barrier()
    pltpu.sync_copy(d_sh.at[pl.ds(row0, ROWS_PER_TILE)],
                    dd.at[pl.ds(pl.multiple_of(cid * NP + row0, ROWS_PER_TILE), ROWS_PER_TILE)])


def _sc_logits(eu, ev, tab):
    mesh = plsc.VectorSubcoreMesh(core_axis_name="c", subcore_axis_name="s",
                                  num_cores=NC, num_subcores=NS)
    f32 = jnp.float32
    return pl.kernel(
        _sc_logits_body,
        out_type=[
            jax.ShapeDtypeStruct((E // 8, 128), f32),
            jax.ShapeDtypeStruct((E // 8, 128), f32),
            jax.ShapeDtypeStruct((NC * NP, OUT), f32),
        ],
        mesh=mesh,
        scratch_types=[
            pltpu.VMEM((B1,), jnp.int32),
            pltpu.VMEM((B1,), jnp.int32),
            pltpu.VMEM((B1, OUT), f32),
            pltpu.VMEM((B1, OUT), f32),
            pltpu.VMEM((B1 // 8, 128), f32),
            pltpu.VMEM((B1 // 8, 128), f32),
            pltpu.VMEM((B1, OUT), f32),
            pltpu.VMEM((B1, OUT), f32),
            pltpu.VMEM_SHARED((NP, OUT), f32),
            pltpu.SemaphoreType.DMA,
        ],
    )(eu, ev, tab)


def _rdenom_body(da, db, rd_ref):
    r1 = 1.0 / (da[:, 0:16] + db[:, 0:16] + 1e-16)
    r2 = 1.0 / (da[:, 16:32] + db[:, 16:32] + 1e-16)
    pad = jnp.zeros((r1.shape[0], OUT - 2 * H), jnp.float32)
    rd_ref[...] = jnp.concatenate([r1, r2, pad], axis=1)


def _rdenom(dd):
    f32 = jnp.float32
    return pl.pallas_call(
        _rdenom_body,
        out_shape=jax.ShapeDtypeStruct((NP, OUT), f32),
    )(dd[:NP], dd[NP:])


CH = 64
NCHG = E // (CH * B2)
TCH = (NCHG + NS - 1) // NS


def _sc_agg_body(eu, ev, ee1, ee2, rd, h1, h2,
                 osum,
                 idxs_c, idxd_c, ee_c, hbufs, rdbufs, msg,
                 o_sh, semh0, semh1, semr0, semr1):
    cid = lax.axis_index("c")
    sid = lax.axis_index("s")
    hbuf = [hbufs.at[0], hbufs.at[1]]
    rdbuf = [rdbufs.at[0], rdbufs.at[1]]
    semh = [semh0, semh1]
    semr = [semr0, semr1]

    @pl.loop(0, 2 * B2)
    def _zm(i):
        for k in range(OUT // 16):
            msg[i, pl.ds(k * 16, 16)] = jnp.zeros((16,), jnp.float32)

    row0 = pl.multiple_of(sid * ROWS_PER_TILE, ROWS_PER_TILE)

    @pl.loop(0, ROWS_PER_TILE // (2 * B2))
    def _zs(q):
        pltpu.sync_copy(msg, o_sh.at[pl.ds(row0 + q * 2 * B2, 2 * B2)])
    plsc.subcore_barrier()

    def run(src_hbm, dst_hbm, ee_hbm, h_hbm, co):
        def issue(b, p):
            pltpu.async_copy(h_hbm.at[idxs_c.at[pl.ds(b * B2, B2)]],
                             hbuf[p], semh[p])
            pltpu.async_copy(rd.at[idxd_c.at[pl.ds(b * B2, B2)]],
                             rdbuf[p], semr[p])

        def drain(p):
            pltpu.make_async_copy(h_hbm.at[pl.ds(0, B2)], hbuf[p], semh[p]).wait()
            pltpu.make_async_copy(rd.at[pl.ds(0, B2)], rdbuf[p], semr[p]).wait()

        def compute(b, p, mrow):
            for i in range(B2):
                a = ee_c[b, pl.ds(i * 16, 16)] * rdbuf[p][i, pl.ds(co, 16)]
                alphas = [a[h] for h in range(H)]
                for k in range(OUT // 16):
                    acc = alphas[0] * hbuf[p][i, pl.ds(k * 16, 16)]
                    for h in range(1, H):
                        acc = acc + alphas[h] * hbuf[p][i, pl.ds(h * OUT + k * 16, 16)]
                    msg[mrow + i, pl.ds(k * 16, 16)] = acc

        @pl.loop(0, TCH)
        def _chunk(c):
            cglob = c * NS + sid

            @pl.when(cglob < NCHG)
            def _():
                base = pl.multiple_of(cglob * CH * B2, CH * B2)
                pltpu.sync_copy(src_hbm.at[pl.ds(base, CH * B2)], idxs_c)
                pltpu.sync_copy(dst_hbm.at[pl.ds(base, CH * B2)], idxd_c)
                pltpu.sync_copy(ee_hbm.at[pl.ds(pl.multiple_of(base // 8, CH), CH)], ee_c)
                issue(0, 0)
                issue(1, 1)

                @pl.loop(0, CH // 2)
                def _pair(t):
                    b0 = t * 2
                    drain(0)
                    compute(b0, 0, 0)
                    @pl.when(b0 + 2 < CH)
                    def _i0():
                        issue(b0 + 2, 0)
                    drain(1)
                    compute(b0 + 1, 1, B2)
                    @pl.when(b0 + 3 < CH)
                    def _i1():
                        issue(b0 + 3, 1)
                    idxv = idxd_c[pl.ds(b0 * B2, 2 * B2)]
                    pltpu.sync_copy(msg, o_sh.at[idxv], add=True)

    @pl.when(cid == 0)
    def _():
        run(eu, ev, ee1, h1, 0)

    @pl.when(cid == 1)
    def _():
        run(ev, eu, ee2, h2, 16)

    plsc.subcore_barrier()
    pltpu.sync_copy(o_sh.at[pl.ds(row0, ROWS_PER_TILE)],
                    osum.at[pl.ds(pl.multiple_of(cid * NP + row0, ROWS_PER_TILE), ROWS_PER_TILE)])


def _sc_agg(eu, ev, ee1, ee2, rd, h1, h2):
    mesh = plsc.VectorSubcoreMesh(core_axis_name="c", subcore_axis_name="s",
                                  num_cores=NC, num_subcores=NS)
    f32 = jnp.float32
    return pl.kernel(
        _sc_agg_body,
        out_type=jax.ShapeDtypeStruct((NC * NP, OUT), f32),
        mesh=mesh,
        scratch_types=[
            pltpu.VMEM((CH * B2,), jnp.int32),
            pltpu.VMEM((CH * B2,), jnp.int32),
            pltpu.VMEM((CH, 128), f32),
            pltpu.VMEM((2, B2, HD), f32),
            pltpu.VMEM((2, B2, OUT), f32),
            pltpu.VMEM((2 * B2, OUT), f32),
            pltpu.VMEM_SHARED((NP, OUT), f32),
            pltpu.SemaphoreType.DMA,
            pltpu.SemaphoreType.DMA,
            pltpu.SemaphoreType.DMA,
            pltpu.SemaphoreType.DMA,
        ],
    )(eu, ev, ee1, ee2, rd, h1, h2)


def _finish_body(o1_ref, o2_ref, b1_ref, b2_ref, xin_ref, xout_ref):
    s = 1.0 / H
    a = o1_ref[...] * s + b1_ref[...]
    xin_ref[...] = jnp.where(a > 0, a, jnp.exp(jnp.minimum(a, 0.0)) - 1.0)
    b = o2_ref[...] * s + b2_ref[...]
    xout_ref[...] = jnp.where(b > 0, b, jnp.exp(jnp.minimum(b, 0.0)) - 1.0)


def _finish(o1, o2, b1, b2):
    RB = 1000
    grid = (N // RB,)
    row = pl.BlockSpec((RB, OUT), lambda i: (i, 0))
    vec = pl.BlockSpec((OUT,), lambda i: (0,))
    return pl.pallas_call(
        _finish_body,
        grid=grid,
        in_specs=[row, row, vec, vec],
        out_specs=[row, row],
        out_shape=[
            jax.ShapeDtypeStruct((N, OUT), jnp.float32),
            jax.ShapeDtypeStruct((N, OUT), jnp.float32),
        ],
    )(o1, o2, b1, b2)


@jax.jit
def kernel(x, edge_index, W1, a_s1, a_d1, b1, W2, a_s2, a_d2, b2,
           lw1, lb1, lw2, lb2):
    eu = edge_index[0]
    ev = edge_index[1]
    W1r = W1.reshape(H, OUT, D)
    W2r = W2.reshape(H, OUT, D)
    as1 = a_s1.reshape(H, OUT)
    ad1 = a_d1.reshape(H, OUT)
    as2 = a_s2.reshape(H, OUT)
    ad2 = a_d2.reshape(H, OUT)

    h1, h2, tab, xs = _dense(x, W1, W1r, as1, ad1, W2, W2r, as2, ad2,
                             lw1, lb1, lw2, lb2)
    ee1, ee2, dd = _sc_logits(eu, ev, tab)
    rd = _rdenom(dd)
    osum = _sc_agg(eu, ev, ee1, ee2, rd, h1, h2)
    x_in, x_out = _finish(osum[:N], osum[NP:NP + N], b1, b2)
    return (x_in, x_out, xs)
